# baseline (speedup 1.0000x reference)
import functools

import jax
import jax.numpy as jnp
from jax import lax
from jax.experimental import pallas as pl
from jax.experimental.pallas import tpu as pltpu
from jax.experimental.pallas import tpu_sc as plsc

N = 10000
E = 320000
D = 128
NC = 2
NS = 16
NW = NC * NS
KC = 128
NCH = 78
EPW = NCH * KC
XTRA = (E - NW * EPW) // KC

RPT = 624
RTAIL = N - NS * RPT

W1ROW = 34
W2ROW = 8

_mesh = plsc.VectorSubcoreMesh(core_axis_name="c", subcore_axis_name="s")


def _leaky(v):
    return jnp.where(v > 0, v, v * jnp.float32(0.2))


def _elu(v):
    return jnp.where(v > 0, v, jnp.exp(v) - jnp.float32(1.0))


def _tc1_body(x_ref, w1_ref, asrc_ref, adst_ref, h1_ref, asad_ref):
    h = jnp.dot(x_ref[...], w1_ref[...], preferred_element_type=jnp.float32)
    h1_ref[...] = h
    as0 = jnp.sum(h[:, 0:16] * asrc_ref[0, :][None, :], axis=1, keepdims=True)
    as1 = jnp.sum(h[:, 16:32] * asrc_ref[1, :][None, :], axis=1, keepdims=True)
    ad0 = jnp.sum(h[:, 0:16] * adst_ref[0, :][None, :], axis=1, keepdims=True)
    ad1 = jnp.sum(h[:, 16:32] * adst_ref[1, :][None, :], axis=1, keepdims=True)
    asad_ref[...] = jnp.concatenate([as0, as1, ad0, ad1], axis=1)


def _tc1(x, W1, att_src1, att_dst1):
    return pl.pallas_call(
        _tc1_body,
        out_shape=[
            jax.ShapeDtypeStruct((N, 32), jnp.float32),
            jax.ShapeDtypeStruct((N, 4), jnp.float32),
        ],
    )(x, W1, att_src1, att_dst1)


def _sc1_body(src_hbm, dst_hbm, h1_hbm, asad_hbm, out_hbm,
              srcv, dstv, gbuf, payv, asadv, shacc, sem):
    cid = lax.axis_index("c")
    sid = lax.axis_index("s")
    wid = sid * NC + cid
    iota16 = lax.iota(jnp.int32, 16)
    zero16 = jnp.zeros((16,), jnp.float32)

    pltpu.sync_copy(asad_hbm, asadv)

    def _zrow(r, carry):
        payv[r, pl.ds(0, 16)] = zero16
        payv[r, pl.ds(16, 16)] = zero16
        payv[r, pl.ds(18, 16)] = zero16
        return carry
    lax.fori_loop(0, KC, _zrow, 0)
    for k in range(4):
        pltpu.sync_copy(payv, shacc.at[pl.ds(sid * RPT + k * KC, KC)])
    pltpu.sync_copy(payv.at[pl.ds(0, RPT - 4 * KC)],
                    shacc.at[pl.ds(sid * RPT + 4 * KC, RPT - 4 * KC)])

    @pl.when(sid == NS - 1)
    def _ztail():
        pltpu.sync_copy(payv.at[pl.ds(0, RTAIL)],
                        shacc.at[pl.ds(NS * RPT, RTAIL)])
    plsc.subcore_barrier()

    c_as0 = jnp.zeros((16,), jnp.int32)
    c_as1 = jnp.full((16,), 1, jnp.int32)
    c_ad0 = jnp.full((16,), 2, jnp.int32)
    c_ad1 = jnp.full((16,), 3, jnp.int32)
    c_w0 = jnp.full((16,), 32, jnp.int32)
    c_w1 = jnp.full((16,), 33, jnp.int32)

    def _do_chunk(base):
        pltpu.sync_copy(src_hbm.at[pl.ds(base, KC)], srcv)
        pltpu.sync_copy(dst_hbm.at[pl.ds(base, KC)], dstv)
        pltpu.async_copy(h1_hbm.at[srcv], gbuf, sem).wait()

        def _vec(j, icarry):
            jb = j * 16
            sidx = srcv[pl.ds(jb, 16)]
            didx = dstv[pl.ds(jb, 16)]
            es0 = plsc.load_gather(asadv, [sidx, c_as0])
            es1 = plsc.load_gather(asadv, [sidx, c_as1])
            ed0 = plsc.load_gather(asadv, [didx, c_ad0])
            ed1 = plsc.load_gather(asadv, [didx, c_ad1])
            w0 = jnp.exp(_leaky(es0 + ed0))
            w1 = jnp.exp(_leaky(es1 + ed1))
            rows = jb + iota16
            plsc.store_scatter(payv, [rows, c_w0], w0)
            plsc.store_scatter(payv, [rows, c_w1], w1)
            for cc in range(16):
                col0 = jnp.full((16,), cc, jnp.int32)
                col1 = jnp.full((16,), cc + 16, jnp.int32)
                g0 = plsc.load_gather(gbuf, [rows, col0])
                g1 = plsc.load_gather(gbuf, [rows, col1])
                plsc.store_scatter(payv, [rows, col0], w0 * g0)
                plsc.store_scatter(payv, [rows, col1], w1 * g1)
            return icarry
        lax.fori_loop(0, KC // 16, _vec, 0)

        pltpu.sync_copy(payv, shacc.at[dstv], add=True)

    def _chunk(c, carry):
        _do_chunk(wid * EPW + c * KC)
        return carry
    lax.fori_loop(0, NCH, _chunk, 0)

    @pl.when(wid < XTRA)
    def _extra():
        _do_chunk(NW * EPW + wid * KC)
    plsc.subcore_barrier()

    pltpu.sync_copy(shacc.at[pl.ds(sid * RPT, RPT)],
                    out_hbm.at[cid, pl.ds(sid * RPT, RPT)])

    @pl.when(sid == NS - 1)
    def _otail():
        pltpu.sync_copy(shacc.at[pl.ds(NS * RPT, RTAIL)],
                        out_hbm.at[cid, pl.ds(NS * RPT, RTAIL)])


@functools.partial(
    pl.kernel,
    out_type=jax.ShapeDtypeStruct((NC, N, W1ROW), jnp.float32),
    mesh=_mesh,
    compiler_params=pltpu.CompilerParams(use_tc_tiling_on_sc=False, needs_layout_passes=False),
    scratch_types=[
        pltpu.VMEM((KC,), jnp.int32),
        pltpu.VMEM((KC,), jnp.int32),
        pltpu.VMEM((KC, 32), jnp.float32),
        pltpu.VMEM((KC, W1ROW), jnp.float32),
        pltpu.VMEM((N, 4), jnp.float32),
        pltpu.VMEM_SHARED((N, W1ROW), jnp.float32),
        pltpu.SemaphoreType.DMA,
    ],
)
def _sc1(src_hbm, dst_hbm, h1_hbm, asad_hbm, out_hbm, *scratch):
    _sc1_body(src_hbm, dst_hbm, h1_hbm, asad_hbm, out_hbm, *scratch)


def _tc2_body(acc_a, acc_b, h1_ref, asad_ref, b1_ref, w2_ref,
              asrc2_ref, adst2_ref, tab2_ref):
    h1 = h1_ref[...]
    asad = asad_ref[...]
    ws0 = jnp.exp(_leaky(asad[:, 0:1] + asad[:, 2:3]))
    ws1 = jnp.exp(_leaky(asad[:, 1:2] + asad[:, 3:4]))
    num0 = acc_a[:, 0:16] + acc_b[:, 0:16] + ws0 * h1[:, 0:16]
    num1 = acc_a[:, 16:32] + acc_b[:, 16:32] + ws1 * h1[:, 16:32]
    den0 = acc_a[:, 32:33] + acc_b[:, 32:33] + ws0
    den1 = acc_a[:, 33:34] + acc_b[:, 33:34] + ws1
    o0 = num0 / (den0 + jnp.float32(1e-16))
    o1 = num1 / (den1 + jnp.float32(1e-16))
    hmid = _elu(jnp.concatenate([o0, o1], axis=1) + b1_ref[...])
    h2 = jnp.dot(hmid, w2_ref[...], preferred_element_type=jnp.float32)
    as2 = jnp.sum(h2 * asrc2_ref[0, :][None, :], axis=1, keepdims=True)
    ad2 = jnp.sum(h2 * adst2_ref[0, :][None, :], axis=1, keepdims=True)
    tab2_ref[...] = jnp.concatenate([h2, as2, ad2], axis=1)


def _tc2(acc_a, acc_b, h1, asad, b1, W2, att_src2, att_dst2):
    return pl.pallas_call(
        _tc2_body,
        out_shape=jax.ShapeDtypeStruct((N, 4), jnp.float32),
    )(acc_a, acc_b, h1, asad, b1, W2, att_src2, att_dst2)


def _sc2_body(src_hbm, dst_hbm, tab_hbm, out_hbm,
              srcv, dstv, payv, tabv, shacc, sem):
    cid = lax.axis_index("c")
    sid = lax.axis_index("s")
    wid = sid * NC + cid
    iota16 = lax.iota(jnp.int32, 16)
    zero16 = jnp.zeros((16,), jnp.float32)

    pltpu.sync_copy(tab_hbm, tabv)

    def _z(i, carry):
        fi = i * 16 + iota16
        plsc.store_scatter(payv, [lax.shift_right_logical(fi, 3),
                                  lax.bitwise_and(fi, 7)], zero16)
        return carry
    lax.fori_loop(0, KC * W2ROW // 16, _z, 0)
    for k in range(4):
        pltpu.sync_copy(payv, shacc.at[pl.ds(sid * RPT + k * KC, KC)])
    pltpu.sync_copy(payv.at[pl.ds(0, RPT - 4 * KC)],
                    shacc.at[pl.ds(sid * RPT + 4 * KC, RPT - 4 * KC)])

    @pl.when(sid == NS - 1)
    def _ztail():
        pltpu.sync_copy(payv.at[pl.ds(0, RTAIL)],
                        shacc.at[pl.ds(NS * RPT, RTAIL)])
    plsc.subcore_barrier()

    c_h0 = jnp.zeros((16,), jnp.int32)
    c_h1 = jnp.full((16,), 1, jnp.int32)
    c_as = jnp.full((16,), 2, jnp.int32)
    c_ad = jnp.full((16,), 3, jnp.int32)
    p_0 = jnp.zeros((16,), jnp.int32)
    p_1 = jnp.full((16,), 1, jnp.int32)
    p_2 = jnp.full((16,), 2, jnp.int32)

    def _do_chunk(base):
        pltpu.sync_copy(src_hbm.at[pl.ds(base, KC)], srcv)
        pltpu.sync_copy(dst_hbm.at[pl.ds(base, KC)], dstv)

        def _vec(j, icarry):
            jb = j * 16
            sidx = srcv[pl.ds(jb, 16)]
            didx = dstv[pl.ds(jb, 16)]
            h20 = plsc.load_gather(tabv, [sidx, c_h0])
            h21 = plsc.load_gather(tabv, [sidx, c_h1])
            as2 = plsc.load_gather(tabv, [sidx, c_as])
            ad2 = plsc.load_gather(tabv, [didx, c_ad])
            w = jnp.exp(_leaky(as2 + ad2))
            rows = jb + iota16
            plsc.store_scatter(payv, [rows, p_0], w * h20)
            plsc.store_scatter(payv, [rows, p_1], w * h21)
            plsc.store_scatter(payv, [rows, p_2], w)
            return icarry
        lax.fori_loop(0, KC // 16, _vec, 0)

        pltpu.sync_copy(payv, shacc.at[dstv], add=True)

    def _chunk(c, carry):
        _do_chunk(wid * EPW + c * KC)
        return carry
    lax.fori_loop(0, NCH, _chunk, 0)

    @pl.when(wid < XTRA)
    def _extra():
        _do_chunk(NW * EPW + wid * KC)
    plsc.subcore_barrier()

    pltpu.sync_copy(shacc.at[pl.ds(sid * RPT, RPT)],
                    out_hbm.at[cid, pl.ds(sid * RPT, RPT)])

    @pl.when(sid == NS - 1)
    def _otail():
        pltpu.sync_copy(shacc.at[pl.ds(NS * RPT, RTAIL)],
                        out_hbm.at[cid, pl.ds(NS * RPT, RTAIL)])


@functools.partial(
    pl.kernel,
    out_type=jax.ShapeDtypeStruct((NC, N, W2ROW), jnp.float32),
    mesh=_mesh,
    compiler_params=pltpu.CompilerParams(use_tc_tiling_on_sc=False, needs_layout_passes=False),
    scratch_types=[
        pltpu.VMEM((KC,), jnp.int32),
        pltpu.VMEM((KC,), jnp.int32),
        pltpu.VMEM((KC, W2ROW), jnp.float32),
        pltpu.VMEM((N, 4), jnp.float32),
        pltpu.VMEM_SHARED((N, W2ROW), jnp.float32),
        pltpu.SemaphoreType.DMA,
    ],
)
def _sc2(src_hbm, dst_hbm, tab_hbm, out_hbm, *scratch):
    _sc2_body(src_hbm, dst_hbm, tab_hbm, out_hbm, *scratch)


def _tc3_body(acc_a, acc_b, tab2_ref, b2_ref, lw1_ref, lb1_ref,
              lw2_ref, lb2_ref, out_ref):
    t2 = tab2_ref[...]
    ws = jnp.exp(_leaky(t2[:, 2:3] + t2[:, 3:4]))
    num = acc_a[:, 0:2] + acc_b[:, 0:2] + ws * t2[:, 0:2]
    den = acc_a[:, 2:3] + acc_b[:, 2:3] + ws
    g = num / (den + jnp.float32(1e-16)) + b2_ref[...]
    t = _elu(jnp.dot(g, lw1_ref[...], preferred_element_type=jnp.float32)
             + lb1_ref[...])
    y = jnp.dot(t, lw2_ref[...], preferred_element_type=jnp.float32) + lb2_ref[...]
    out_ref[...] = jnp.mean(y, axis=0, keepdims=True)


def _tc3(acc_a, acc_b, tab2, b2, lw1, lb1, lw2, lb2):
    return pl.pallas_call(
        _tc3_body,
        out_shape=jax.ShapeDtypeStruct((1, 2), jnp.float32),
    )(acc_a, acc_b, tab2, b2, lw1, lb1, lw2, lb2)


def kernel(x, edge_index, W1, att_src1, att_dst1, b1, W2, att_src2,
           att_dst2, b2, lw1, lb1, lw2, lb2):
    src = edge_index[0]
    dst = edge_index[1]
    h1, asad1 = _tc1(x, W1, att_src1, att_dst1)
    acc1 = _sc1(src, dst, h1, asad1)
    tab2 = _tc2(acc1[0], acc1[1], h1, asad1, b1.reshape(1, -1), W2,
                att_src2, att_dst2)
    acc2 = _sc2(src, dst, tab2)
    return _tc3(acc2[0], acc2[1], tab2, b2.reshape(1, -1), lw1,
                lb1.reshape(1, -1), lw2, lb2.reshape(1, -1))


# speedup vs baseline: 1.0330x; 1.0330x over previous
import functools

import jax
import jax.numpy as jnp
from jax import lax
from jax.experimental import pallas as pl
from jax.experimental.pallas import tpu as pltpu
from jax.experimental.pallas import tpu_sc as plsc

N = 10000
E = 320000
D = 128
NC = 2
NS = 16
NW = NC * NS
KC = 128
NCH = 78
EPW = NCH * KC
XTRA = (E - NW * EPW) // KC

RPT = 624
RTAIL = N - NS * RPT

W1ROW = 34
W2ROW = 8

_mesh = plsc.VectorSubcoreMesh(core_axis_name="c", subcore_axis_name="s")


def _leaky(v):
    return jnp.where(v > 0, v, v * jnp.float32(0.2))


def _elu(v):
    return jnp.where(v > 0, v, jnp.exp(v) - jnp.float32(1.0))


def _tc1_body(x_ref, w1_ref, asrc_ref, adst_ref, h1_ref, asad_ref):
    h = jnp.dot(x_ref[...], w1_ref[...], preferred_element_type=jnp.float32)
    h1_ref[...] = h
    as0 = jnp.sum(h[:, 0:16] * asrc_ref[0, :][None, :], axis=1, keepdims=True)
    as1 = jnp.sum(h[:, 16:32] * asrc_ref[1, :][None, :], axis=1, keepdims=True)
    ad0 = jnp.sum(h[:, 0:16] * adst_ref[0, :][None, :], axis=1, keepdims=True)
    ad1 = jnp.sum(h[:, 16:32] * adst_ref[1, :][None, :], axis=1, keepdims=True)
    asad_ref[...] = jnp.concatenate([as0, as1, ad0, ad1], axis=1)


def _tc1(x, W1, att_src1, att_dst1):
    return pl.pallas_call(
        _tc1_body,
        out_shape=[
            jax.ShapeDtypeStruct((N, 32), jnp.float32),
            jax.ShapeDtypeStruct((N, 4), jnp.float32),
        ],
    )(x, W1, att_src1, att_dst1)


def _sc1_body(src_hbm, dst_hbm, h1_hbm, asad_hbm, out_hbm,
              srcv_a, dstv_a, gbuf_a, payv_a,
              srcv_b, dstv_b, gbuf_b, payv_b,
              asadv, shacc, gsem, sem_a, sem_b):
    cid = lax.axis_index("c")
    sid = lax.axis_index("s")
    wid = sid * NC + cid
    iota16 = lax.iota(jnp.int32, 16)
    zero16 = jnp.zeros((16,), jnp.float32)

    pltpu.sync_copy(asad_hbm, asadv)

    def _zrow(r, carry):
        payv_a[r, pl.ds(0, 16)] = zero16
        payv_a[r, pl.ds(16, 16)] = zero16
        payv_a[r, pl.ds(18, 16)] = zero16
        return carry
    lax.fori_loop(0, KC, _zrow, 0)
    for k in range(4):
        pltpu.sync_copy(payv_a, shacc.at[pl.ds(sid * RPT + k * KC, KC)])
    pltpu.sync_copy(payv_a.at[pl.ds(0, RPT - 4 * KC)],
                    shacc.at[pl.ds(sid * RPT + 4 * KC, RPT - 4 * KC)])

    @pl.when(sid == NS - 1)
    def _ztail():
        pltpu.sync_copy(payv_a.at[pl.ds(0, RTAIL)],
                        shacc.at[pl.ds(NS * RPT, RTAIL)])
    plsc.subcore_barrier()

    c_as0 = jnp.zeros((16,), jnp.int32)
    c_as1 = jnp.full((16,), 1, jnp.int32)
    c_ad0 = jnp.full((16,), 2, jnp.int32)
    c_ad1 = jnp.full((16,), 3, jnp.int32)
    c_w0 = jnp.full((16,), 32, jnp.int32)
    c_w1 = jnp.full((16,), 33, jnp.int32)

    def _fill(base, srcv, dstv, gbuf, payv):
        pltpu.sync_copy(src_hbm.at[pl.ds(base, KC)], srcv)
        pltpu.sync_copy(dst_hbm.at[pl.ds(base, KC)], dstv)
        pltpu.async_copy(h1_hbm.at[srcv], gbuf, gsem).wait()

        def _vec(j, icarry):
            jb = j * 16
            sidx = srcv[pl.ds(jb, 16)]
            didx = dstv[pl.ds(jb, 16)]
            es0 = plsc.load_gather(asadv, [sidx, c_as0])
            es1 = plsc.load_gather(asadv, [sidx, c_as1])
            ed0 = plsc.load_gather(asadv, [didx, c_ad0])
            ed1 = plsc.load_gather(asadv, [didx, c_ad1])
            w0 = jnp.exp(_leaky(es0 + ed0))
            w1 = jnp.exp(_leaky(es1 + ed1))
            rows = jb + iota16
            plsc.store_scatter(payv, [rows, c_w0], w0)
            plsc.store_scatter(payv, [rows, c_w1], w1)
            for cc in range(16):
                col0 = jnp.full((16,), cc, jnp.int32)
                col1 = jnp.full((16,), cc + 16, jnp.int32)
                g0 = plsc.load_gather(gbuf, [rows, col0])
                g1 = plsc.load_gather(gbuf, [rows, col1])
                plsc.store_scatter(payv, [rows, col0], w0 * g0)
                plsc.store_scatter(payv, [rows, col1], w1 * g1)
            return icarry
        lax.fori_loop(0, KC // 16, _vec, 0)

    def _pair(i, carry):
        @pl.when(i > 0)
        def _wa():
            pltpu.make_async_copy(payv_a, shacc.at[dstv_a], sem_a).wait()
        _fill(wid * EPW + (2 * i) * KC, srcv_a, dstv_a, gbuf_a, payv_a)

        @pl.when(i > 0)
        def _wb():
            pltpu.make_async_copy(payv_b, shacc.at[dstv_b], sem_b).wait()
        pltpu.async_copy(payv_a, shacc.at[dstv_a], sem_a, add=True)
        _fill(wid * EPW + (2 * i + 1) * KC, srcv_b, dstv_b, gbuf_b, payv_b)
        pltpu.async_copy(payv_b, shacc.at[dstv_b], sem_b, add=True)
        return carry
    lax.fori_loop(0, NCH // 2, _pair, 0)
    pltpu.make_async_copy(payv_a, shacc.at[dstv_a], sem_a).wait()
    pltpu.make_async_copy(payv_b, shacc.at[dstv_b], sem_b).wait()

    @pl.when(wid < XTRA)
    def _extra():
        _fill(NW * EPW + wid * KC, srcv_a, dstv_a, gbuf_a, payv_a)
        pltpu.sync_copy(payv_a, shacc.at[dstv_a], add=True)
    plsc.subcore_barrier()

    pltpu.sync_copy(shacc.at[pl.ds(sid * RPT, RPT)],
                    out_hbm.at[cid, pl.ds(sid * RPT, RPT)])

    @pl.when(sid == NS - 1)
    def _otail():
        pltpu.sync_copy(shacc.at[pl.ds(NS * RPT, RTAIL)],
                        out_hbm.at[cid, pl.ds(NS * RPT, RTAIL)])


@functools.partial(
    pl.kernel,
    out_type=jax.ShapeDtypeStruct((NC, N, W1ROW), jnp.float32),
    mesh=_mesh,
    compiler_params=pltpu.CompilerParams(use_tc_tiling_on_sc=False, needs_layout_passes=False),
    scratch_types=[
        pltpu.VMEM((KC,), jnp.int32),
        pltpu.VMEM((KC,), jnp.int32),
        pltpu.VMEM((KC, 32), jnp.float32),
        pltpu.VMEM((KC, W1ROW), jnp.float32),
        pltpu.VMEM((KC,), jnp.int32),
        pltpu.VMEM((KC,), jnp.int32),
        pltpu.VMEM((KC, 32), jnp.float32),
        pltpu.VMEM((KC, W1ROW), jnp.float32),
        pltpu.VMEM((N, 4), jnp.float32),
        pltpu.VMEM_SHARED((N, W1ROW), jnp.float32),
        pltpu.SemaphoreType.DMA,
        pltpu.SemaphoreType.DMA,
        pltpu.SemaphoreType.DMA,
    ],
)
def _sc1(src_hbm, dst_hbm, h1_hbm, asad_hbm, out_hbm, *scratch):
    _sc1_body(src_hbm, dst_hbm, h1_hbm, asad_hbm, out_hbm, *scratch)


def _tc2_body(acc_a, acc_b, h1_ref, asad_ref, b1_ref, w2_ref,
              asrc2_ref, adst2_ref, tab2_ref):
    h1 = h1_ref[...]
    asad = asad_ref[...]
    ws0 = jnp.exp(_leaky(asad[:, 0:1] + asad[:, 2:3]))
    ws1 = jnp.exp(_leaky(asad[:, 1:2] + asad[:, 3:4]))
    num0 = acc_a[:, 0:16] + acc_b[:, 0:16] + ws0 * h1[:, 0:16]
    num1 = acc_a[:, 16:32] + acc_b[:, 16:32] + ws1 * h1[:, 16:32]
    den0 = acc_a[:, 32:33] + acc_b[:, 32:33] + ws0
    den1 = acc_a[:, 33:34] + acc_b[:, 33:34] + ws1
    o0 = num0 / (den0 + jnp.float32(1e-16))
    o1 = num1 / (den1 + jnp.float32(1e-16))
    hmid = _elu(jnp.concatenate([o0, o1], axis=1) + b1_ref[...])
    h2 = jnp.dot(hmid, w2_ref[...], preferred_element_type=jnp.float32)
    as2 = jnp.sum(h2 * asrc2_ref[0, :][None, :], axis=1, keepdims=True)
    ad2 = jnp.sum(h2 * adst2_ref[0, :][None, :], axis=1, keepdims=True)
    tab2_ref[...] = jnp.concatenate([h2, as2, ad2], axis=1)


def _tc2(acc_a, acc_b, h1, asad, b1, W2, att_src2, att_dst2):
    return pl.pallas_call(
        _tc2_body,
        out_shape=jax.ShapeDtypeStruct((N, 4), jnp.float32),
    )(acc_a, acc_b, h1, asad, b1, W2, att_src2, att_dst2)


def _sc2_body(src_hbm, dst_hbm, tab_hbm, out_hbm,
              srcv, dstv, payv, tabv, shacc, sem):
    cid = lax.axis_index("c")
    sid = lax.axis_index("s")
    wid = sid * NC + cid
    iota16 = lax.iota(jnp.int32, 16)
    zero16 = jnp.zeros((16,), jnp.float32)

    pltpu.sync_copy(tab_hbm, tabv)

    def _z(i, carry):
        fi = i * 16 + iota16
        plsc.store_scatter(payv, [lax.shift_right_logical(fi, 3),
                                  lax.bitwise_and(fi, 7)], zero16)
        return carry
    lax.fori_loop(0, KC * W2ROW // 16, _z, 0)
    for k in range(4):
        pltpu.sync_copy(payv, shacc.at[pl.ds(sid * RPT + k * KC, KC)])
    pltpu.sync_copy(payv.at[pl.ds(0, RPT - 4 * KC)],
                    shacc.at[pl.ds(sid * RPT + 4 * KC, RPT - 4 * KC)])

    @pl.when(sid == NS - 1)
    def _ztail():
        pltpu.sync_copy(payv.at[pl.ds(0, RTAIL)],
                        shacc.at[pl.ds(NS * RPT, RTAIL)])
    plsc.subcore_barrier()

    c_h0 = jnp.zeros((16,), jnp.int32)
    c_h1 = jnp.full((16,), 1, jnp.int32)
    c_as = jnp.full((16,), 2, jnp.int32)
    c_ad = jnp.full((16,), 3, jnp.int32)
    p_0 = jnp.zeros((16,), jnp.int32)
    p_1 = jnp.full((16,), 1, jnp.int32)
    p_2 = jnp.full((16,), 2, jnp.int32)

    def _do_chunk(base):
        pltpu.sync_copy(src_hbm.at[pl.ds(base, KC)], srcv)
        pltpu.sync_copy(dst_hbm.at[pl.ds(base, KC)], dstv)

        def _vec(j, icarry):
            jb = j * 16
            sidx = srcv[pl.ds(jb, 16)]
            didx = dstv[pl.ds(jb, 16)]
            h20 = plsc.load_gather(tabv, [sidx, c_h0])
            h21 = plsc.load_gather(tabv, [sidx, c_h1])
            as2 = plsc.load_gather(tabv, [sidx, c_as])
            ad2 = plsc.load_gather(tabv, [didx, c_ad])
            w = jnp.exp(_leaky(as2 + ad2))
            rows = jb + iota16
            plsc.store_scatter(payv, [rows, p_0], w * h20)
            plsc.store_scatter(payv, [rows, p_1], w * h21)
            plsc.store_scatter(payv, [rows, p_2], w)
            return icarry
        lax.fori_loop(0, KC // 16, _vec, 0)

        pltpu.sync_copy(payv, shacc.at[dstv], add=True)

    def _chunk(c, carry):
        _do_chunk(wid * EPW + c * KC)
        return carry
    lax.fori_loop(0, NCH, _chunk, 0)

    @pl.when(wid < XTRA)
    def _extra():
        _do_chunk(NW * EPW + wid * KC)
    plsc.subcore_barrier()

    pltpu.sync_copy(shacc.at[pl.ds(sid * RPT, RPT)],
                    out_hbm.at[cid, pl.ds(sid * RPT, RPT)])

    @pl.when(sid == NS - 1)
    def _otail():
        pltpu.sync_copy(shacc.at[pl.ds(NS * RPT, RTAIL)],
                        out_hbm.at[cid, pl.ds(NS * RPT, RTAIL)])


@functools.partial(
    pl.kernel,
    out_type=jax.ShapeDtypeStruct((NC, N, W2ROW), jnp.float32),
    mesh=_mesh,
    compiler_params=pltpu.CompilerParams(use_tc_tiling_on_sc=False, needs_layout_passes=False),
    scratch_types=[
        pltpu.VMEM((KC,), jnp.int32),
        pltpu.VMEM((KC,), jnp.int32),
        pltpu.VMEM((KC, W2ROW), jnp.float32),
        pltpu.VMEM((N, 4), jnp.float32),
        pltpu.VMEM_SHARED((N, W2ROW), jnp.float32),
        pltpu.SemaphoreType.DMA,
    ],
)
def _sc2(src_hbm, dst_hbm, tab_hbm, out_hbm, *scratch):
    _sc2_body(src_hbm, dst_hbm, tab_hbm, out_hbm, *scratch)


def _tc3_body(acc_a, acc_b, tab2_ref, b2_ref, lw1_ref, lb1_ref,
              lw2_ref, lb2_ref, out_ref):
    t2 = tab2_ref[...]
    ws = jnp.exp(_leaky(t2[:, 2:3] + t2[:, 3:4]))
    num = acc_a[:, 0:2] + acc_b[:, 0:2] + ws * t2[:, 0:2]
    den = acc_a[:, 2:3] + acc_b[:, 2:3] + ws
    g = num / (den + jnp.float32(1e-16)) + b2_ref[...]
    t = _elu(jnp.dot(g, lw1_ref[...], preferred_element_type=jnp.float32)
             + lb1_ref[...])
    y = jnp.dot(t, lw2_ref[...], preferred_element_type=jnp.float32) + lb2_ref[...]
    out_ref[...] = jnp.mean(y, axis=0, keepdims=True)


def _tc3(acc_a, acc_b, tab2, b2, lw1, lb1, lw2, lb2):
    return pl.pallas_call(
        _tc3_body,
        out_shape=jax.ShapeDtypeStruct((1, 2), jnp.float32),
    )(acc_a, acc_b, tab2, b2, lw1, lb1, lw2, lb2)


def kernel(x, edge_index, W1, att_src1, att_dst1, b1, W2, att_src2,
           att_dst2, b2, lw1, lb1, lw2, lb2):
    src = edge_index[0]
    dst = edge_index[1]
    h1, asad1 = _tc1(x, W1, att_src1, att_dst1)
    acc1 = _sc1(src, dst, h1, asad1)
    tab2 = _tc2(acc1[0], acc1[1], h1, asad1, b1.reshape(1, -1), W2,
                att_src2, att_dst2)
    acc2 = _sc2(src, dst, tab2)
    return _tc3(acc2[0], acc2[1], tab2, b2.reshape(1, -1), lw1,
                lb1.reshape(1, -1), lw2, lb2.reshape(1, -1))
